# baseline (speedup 1.0000x reference)
import jax
import jax.numpy as jnp
from jax.experimental import pallas as pl

NUM_EXPERTS = 64
TOPK = 8
TOKEN_BLOCK = 1024
SUB = 512
NSUB = TOKEN_BLOCK // SUB


def _epilogue(logits, probs_ref, map_ref, s):
    col_max = jnp.max(logits, axis=0, keepdims=True)
    e = jnp.exp(logits - col_max)
    p = e / jnp.sum(e, axis=0, keepdims=True)

    neg_inf = jnp.float32(-jnp.inf)
    rows = jax.lax.broadcasted_iota(jnp.int32, logits.shape, 0)
    v = p
    mask = jnp.zeros(logits.shape, dtype=jnp.bool_)
    for _ in range(TOPK):
        mx = jnp.max(v, axis=0, keepdims=True)
        at_max = v == mx
        first = jnp.min(jnp.where(at_max, rows, NUM_EXPERTS), axis=0,
                        keepdims=True)
        sel = rows == first
        mask = jnp.logical_or(mask, sel)
        v = jnp.where(sel, neg_inf, v)

    probs_ref[pl.ds(s * SUB, SUB), :] = jnp.where(mask, p, 0.0).T
    map_ref[pl.ds(s * SUB, SUB), :] = mask.T.astype(jnp.int8)


def _router_kernel(x_ref, w_ref, probs_ref, map_ref):
    w = w_ref[...]

    def mm(s):
        return jax.lax.dot_general(
            w, x_ref[pl.ds(s * SUB, SUB), :],
            dimension_numbers=(((1,), (1,)), ((), ())),
            preferred_element_type=jnp.float32,
        )

    prev = mm(0)
    for s in range(1, NSUB):
        cur = mm(s)
        _epilogue(prev, probs_ref, map_ref, s - 1)
        prev = cur
    _epilogue(prev, probs_ref, map_ref, NSUB - 1)


@jax.jit
def kernel(x, weight):
    num_tokens, hidden = x.shape
    grid = (num_tokens // TOKEN_BLOCK,)
    probs, rmap = pl.pallas_call(
        _router_kernel,
        grid=grid,
        in_specs=[
            pl.BlockSpec((TOKEN_BLOCK, hidden), lambda i: (i, 0)),
            pl.BlockSpec((NUM_EXPERTS, hidden), lambda i: (0, 0)),
        ],
        out_specs=[
            pl.BlockSpec((TOKEN_BLOCK, NUM_EXPERTS), lambda i: (i, 0)),
            pl.BlockSpec((TOKEN_BLOCK, NUM_EXPERTS), lambda i: (i, 0)),
        ],
        out_shape=[
            jax.ShapeDtypeStruct((num_tokens, NUM_EXPERTS), jnp.float32),
            jax.ShapeDtypeStruct((num_tokens, NUM_EXPERTS), jnp.int8),
        ],
    )(x, weight)
    return probs, rmap.astype(jnp.bool_)


# speedup vs baseline: 1.0259x; 1.0259x over previous
import jax
import jax.numpy as jnp
from jax.experimental import pallas as pl

NUM_EXPERTS = 64
TOPK = 8
TOKEN_BLOCK = 1024
SUB = 128
NSUB = TOKEN_BLOCK // SUB


def _epilogue(logits, probs_ref, map_ref, s):
    col_max = jnp.max(logits, axis=0, keepdims=True)
    e = jnp.exp(logits - col_max)
    p = e / jnp.sum(e, axis=0, keepdims=True)

    neg_inf = jnp.float32(-jnp.inf)
    rows = jax.lax.broadcasted_iota(jnp.int32, logits.shape, 0)
    v = p
    mask = jnp.zeros(logits.shape, dtype=jnp.bool_)
    for _ in range(TOPK):
        mx = jnp.max(v, axis=0, keepdims=True)
        at_max = v == mx
        first = jnp.min(jnp.where(at_max, rows, NUM_EXPERTS), axis=0,
                        keepdims=True)
        sel = rows == first
        mask = jnp.logical_or(mask, sel)
        v = jnp.where(sel, neg_inf, v)

    probs_ref[pl.ds(s * SUB, SUB), :] = jnp.where(mask, p, 0.0).T
    map_ref[pl.ds(s * SUB, SUB), :] = mask.T.astype(jnp.int8)


def _router_kernel(x_ref, w_ref, probs_ref, map_ref):
    w = w_ref[...]

    def mm(s):
        return jax.lax.dot_general(
            w, x_ref[pl.ds(s * SUB, SUB), :],
            dimension_numbers=(((1,), (1,)), ((), ())),
            preferred_element_type=jnp.float32,
        )

    prev = mm(0)
    for s in range(1, NSUB):
        cur = mm(s)
        _epilogue(prev, probs_ref, map_ref, s - 1)
        prev = cur
    _epilogue(prev, probs_ref, map_ref, NSUB - 1)


@jax.jit
def kernel(x, weight):
    num_tokens, hidden = x.shape
    grid = (num_tokens // TOKEN_BLOCK,)
    probs, rmap = pl.pallas_call(
        _router_kernel,
        grid=grid,
        in_specs=[
            pl.BlockSpec((TOKEN_BLOCK, hidden), lambda i: (i, 0)),
            pl.BlockSpec((NUM_EXPERTS, hidden), lambda i: (0, 0)),
        ],
        out_specs=[
            pl.BlockSpec((TOKEN_BLOCK, NUM_EXPERTS), lambda i: (i, 0)),
            pl.BlockSpec((TOKEN_BLOCK, NUM_EXPERTS), lambda i: (i, 0)),
        ],
        out_shape=[
            jax.ShapeDtypeStruct((num_tokens, NUM_EXPERTS), jnp.float32),
            jax.ShapeDtypeStruct((num_tokens, NUM_EXPERTS), jnp.int8),
        ],
    )(x, weight)
    return probs, rmap.astype(jnp.bool_)
